# baseline (speedup 1.0000x reference)
import functools

import jax
import jax.numpy as jnp
from jax import lax
from jax.experimental import pallas as pl
from jax.experimental.pallas import tpu as pltpu
from jax.experimental.pallas import tpu_sc as plsc

NUM_WORKERS = 32
BLOCK_ROWS = 64
NBUF = 8
GLEAD = 4


def _embed_lookup(idx_flat, token_embed, pos_seq):
    n = idx_flat.shape[0]
    seq, d = pos_seq.shape
    per_w = n // NUM_WORKERS
    nblocks = per_w // BLOCK_ROWS
    blocks_per_seq = seq // BLOCK_ROWS

    mesh = plsc.VectorSubcoreMesh(core_axis_name="c", subcore_axis_name="s")

    @functools.partial(
        pl.kernel,
        mesh=mesh,
        out_type=jax.ShapeDtypeStruct((n, d), jnp.float32),
        scratch_types=[
            pltpu.VMEM((per_w,), jnp.int32),
            pltpu.VMEM((seq,), jnp.int32),
            pltpu.VMEM_SHARED((seq, d), jnp.float32),
            pltpu.VMEM_SHARED((16 * BLOCK_ROWS, d), jnp.float32),
        ] + [pltpu.VMEM((BLOCK_ROWS, d), jnp.float32)] * NBUF
          + [pltpu.SemaphoreType.DMA] * NBUF,
    )
    def k(idx_hbm, tok_hbm, pos_hbm, iota_hbm, out_hbm, idx_v, iota_v, pos_s,
          stage_s, *rest):
        rows = rest[:NBUF]
        sems = rest[NBUF:]
        sid = lax.axis_index("s")
        wid = sid * 2 + lax.axis_index("c")
        base = wid * per_w
        pltpu.sync_copy(idx_hbm.at[pl.ds(base, per_w)], idx_v)
        pltpu.sync_copy(iota_hbm.at[pl.ds(0, seq)], iota_v)

        @pl.when(sid == 0)
        def _():
            pltpu.sync_copy(pos_hbm.at[pl.ds(0, seq)], pos_s)

        plsc.subcore_barrier()

        def g_issue(blk, b):
            idx_sl = idx_v.at[pl.ds(blk * BLOCK_ROWS, BLOCK_ROWS)]
            pltpu.async_copy(tok_hbm.at[idx_sl], rows[b], sems[b])

        def drain(b):
            pltpu.make_async_copy(
                tok_hbm.at[pl.ds(0, BLOCK_ROWS)], rows[b], sems[b]).wait()

        def a_issue(b):
            s0 = (b % blocks_per_seq) * BLOCK_ROWS
            pltpu.async_copy(
                pos_s.at[iota_v.at[pl.ds(s0, BLOCK_ROWS)]], rows[b], sems[b],
                add=True)

        def s_issue(blk, b):
            pltpu.async_copy(
                rows[b], stage_s.at[pl.ds(sid * BLOCK_ROWS, BLOCK_ROWS)],
                sems[b])

        def step(t, b, do_gather):
            drain(b)
            a_issue(b)
            b1 = (b - 1) % NBUF
            drain(b1)
            s_issue(t - 1, b1)
            bg = (b + GLEAD) % NBUF
            drain(bg)
            if do_gather:
                g_issue(t + GLEAD, bg)

        for b in range(GLEAD):
            g_issue(b, b)
        drain(0); a_issue(0); g_issue(GLEAD, GLEAD)
        for t in range(1, NBUF - GLEAD):
            drain(t); a_issue(t)
            drain(t - 1); s_issue(t - 1, t - 1)
            g_issue(t + GLEAD, t + GLEAD)
        for t in range(NBUF - GLEAD, NBUF):
            step(t, t, True)

        def outer(g, carry):
            t0 = g * NBUF
            for b in range(NBUF):
                step(t0 + b, b, True)
            return carry

        lax.fori_loop(1, nblocks // NBUF - 1, outer, 0)

        t0 = nblocks - NBUF
        for b in range(NBUF):
            step(t0 + b, b, t0 + b <= nblocks - 1 - GLEAD)

        b_last = (nblocks - 1) % NBUF
        drain(b_last)
        s_issue(nblocks - 1, b_last)
        for u in range(nblocks - NBUF + GLEAD, nblocks - 1):
            drain(u % NBUF)
        drain(b_last)

    iota = jnp.arange(seq, dtype=jnp.int32)
    return k(idx_flat, token_embed, pos_seq, iota)


def kernel(input_Seq, token_embed, pos_embed):
    b, s = input_Seq.shape
    d = token_embed.shape[1]
    idx_flat = input_Seq.reshape(b * s).astype(jnp.int32)
    out_flat = _embed_lookup(idx_flat, token_embed, pos_embed[:s])
    return out_flat.reshape(b, s, d)


# speedup vs baseline: 1.0036x; 1.0036x over previous
import functools

import jax
import jax.numpy as jnp
from jax import lax
from jax.experimental import pallas as pl
from jax.experimental.pallas import tpu as pltpu
from jax.experimental.pallas import tpu_sc as plsc

NUM_WORKERS = 32
BLOCK_ROWS = 64
NBUF = 8
GLEAD = 4


def _embed_lookup(idx_flat, token_embed, pos_seq):
    n = idx_flat.shape[0]
    seq, d = pos_seq.shape
    per_w = n // NUM_WORKERS
    nblocks = per_w // BLOCK_ROWS
    blocks_per_seq = seq // BLOCK_ROWS

    mesh = plsc.VectorSubcoreMesh(core_axis_name="c", subcore_axis_name="s")

    @functools.partial(
        pl.kernel,
        mesh=mesh,
        out_type=jax.ShapeDtypeStruct((n, d), jnp.float32),
        scratch_types=[
            pltpu.VMEM((per_w,), jnp.int32),
            pltpu.VMEM((seq,), jnp.int32),
            pltpu.VMEM_SHARED((seq, d), jnp.float32),
        ] + [pltpu.VMEM((BLOCK_ROWS, d), jnp.float32)] * NBUF
          + [pltpu.SemaphoreType.DMA] * NBUF,
    )
    def k(idx_hbm, tok_hbm, pos_hbm, iota_hbm, out_hbm, idx_v, iota_v, pos_s,
          *rest):
        rows = rest[:NBUF]
        sems = rest[NBUF:]
        sid = lax.axis_index("s")
        wid = sid * 2 + lax.axis_index("c")
        base = wid * per_w
        pltpu.sync_copy(idx_hbm.at[pl.ds(base, per_w)], idx_v)
        pltpu.sync_copy(iota_hbm.at[pl.ds(0, seq)], iota_v)

        @pl.when(sid == 0)
        def _():
            pltpu.sync_copy(pos_hbm.at[pl.ds(0, seq)], pos_s)

        plsc.subcore_barrier()

        def g_issue(blk, b):
            idx_sl = idx_v.at[pl.ds(blk * BLOCK_ROWS, BLOCK_ROWS)]
            pltpu.async_copy(tok_hbm.at[idx_sl], rows[b], sems[b])

        def drain(b):
            pltpu.make_async_copy(
                tok_hbm.at[pl.ds(0, BLOCK_ROWS)], rows[b], sems[b]).wait()

        def a_issue(b):
            s0 = (b % blocks_per_seq) * BLOCK_ROWS
            pltpu.async_copy(
                pos_s.at[iota_v.at[pl.ds(s0, BLOCK_ROWS)]], rows[b], sems[b],
                add=True)

        def s_issue(blk, b):
            pltpu.async_copy(
                rows[b], out_hbm.at[pl.ds(base + blk * BLOCK_ROWS, BLOCK_ROWS)],
                sems[b])

        def step(t, b, do_gather):
            drain(b)
            a_issue(b)
            b1 = (b - 1) % NBUF
            drain(b1)
            s_issue(t - 1, b1)
            bg = (b + GLEAD) % NBUF
            drain(bg)
            if do_gather:
                g_issue(t + GLEAD, bg)

        for b in range(GLEAD):
            g_issue(b, b)
        drain(0); a_issue(0); g_issue(GLEAD, GLEAD)
        for t in range(1, NBUF - GLEAD):
            drain(t); a_issue(t)
            drain(t - 1); s_issue(t - 1, t - 1)
            g_issue(t + GLEAD, t + GLEAD)
        for t in range(NBUF - GLEAD, NBUF):
            step(t, t, True)

        def outer(g, carry):
            t0 = g * NBUF
            for b in range(NBUF):
                step(t0 + b, b, True)
            return carry

        lax.fori_loop(1, nblocks // NBUF - 1, outer, 0)

        t0 = nblocks - NBUF
        for b in range(NBUF):
            step(t0 + b, b, t0 + b <= nblocks - 1 - GLEAD)

        b_last = (nblocks - 1) % NBUF
        drain(b_last)
        s_issue(nblocks - 1, b_last)
        for u in range(nblocks - NBUF + GLEAD, nblocks - 1):
            drain(u % NBUF)
        drain(b_last)

    iota = jnp.arange(seq, dtype=jnp.int32)
    return k(idx_flat, token_embed, pos_seq, iota)


def kernel(input_Seq, token_embed, pos_embed):
    b, s = input_Seq.shape
    d = token_embed.shape[1]
    idx_flat = input_Seq.reshape(b * s).astype(jnp.int32)
    out_flat = _embed_lookup(idx_flat, token_embed, pos_embed[:s])
    return out_flat.reshape(b, s, d)
